# baseline (speedup 1.0000x reference)
import functools
import math

import jax
import jax.numpy as jnp
import numpy as np
from jax import lax
from jax.experimental import pallas as pl
from jax.experimental.pallas import tpu as pltpu
from jax.experimental.pallas import tpu_sc as plsc

_N = 10000
_G = 64
_RATIO = 0.8
_H = 128
_BN_SQRT = np.float32(np.sqrt(1.0 + 1e-5))

_ROW_BLK = 1000


def _proj_body(h_ref, wl_ref, wr_ref, xl_ref, xr_ref):
    h = h_ref[...]
    xl_ref[...] = jnp.dot(h, wl_ref[...], preferred_element_type=jnp.float32)
    xr_ref[...] = jnp.dot(h, wr_ref[...], preferred_element_type=jnp.float32)


def _proj(h, Wl, Wr):
    n = h.shape[0]
    grid = n // _ROW_BLK
    return pl.pallas_call(
        _proj_body,
        grid=(grid,),
        in_specs=[
            pl.BlockSpec((_ROW_BLK, _H), lambda i: (i, 0)),
            pl.BlockSpec((_H, _H), lambda i: (0, 0)),
            pl.BlockSpec((_H, _H), lambda i: (0, 0)),
        ],
        out_specs=[
            pl.BlockSpec((_ROW_BLK, _H), lambda i: (i, 0)),
            pl.BlockSpec((_ROW_BLK, _H), lambda i: (i, 0)),
        ],
        out_shape=[jax.ShapeDtypeStruct((n, _H), jnp.float32)] * 2,
    )(h, Wl, Wr)


def _combine_body(agg_ref, den_ref, xl_ref, xr_ref,
                  att_ref, bias_ref, g_ref, b_ref, wn_ref, nrm_ref,
                  h_ref, score_ref):
    xl = xl_ref[...]
    xr = xr_ref[...]
    m = xl + xr
    m = jnp.where(m >= 0, m, 0.2 * m)
    e0 = jnp.dot(m, att_ref[...], preferred_element_type=jnp.float32)
    ee0 = jnp.exp(e0)
    out = agg_ref[...] + ee0 * xl
    den = den_ref[...] + ee0
    h = out / den + bias_ref[...]
    h = g_ref[...] * (h / _BN_SQRT) + b_ref[...]
    h = jnp.maximum(h, 0.0)
    h_ref[...] = h
    score_ref[...] = jnp.tanh(
        jnp.dot(h, wn_ref[...], preferred_element_type=jnp.float32)
        / nrm_ref[0, 0])


def _combine(agg, den, xl, xr, att, bias, g, b, wn):
    n = xl.shape[0]
    grid = n // _ROW_BLK
    vec = lambda i: (i, 0)
    fixed = lambda i: (0, 0)
    return pl.pallas_call(
        _combine_body,
        grid=(grid,),
        in_specs=[
            pl.BlockSpec((_ROW_BLK, _H), vec),
            pl.BlockSpec((_ROW_BLK, 1), vec),
            pl.BlockSpec((_ROW_BLK, _H), vec),
            pl.BlockSpec((_ROW_BLK, _H), vec),
            pl.BlockSpec((_H, 1), fixed),
            pl.BlockSpec((1, _H), fixed),
            pl.BlockSpec((1, _H), fixed),
            pl.BlockSpec((1, _H), fixed),
            pl.BlockSpec((_H, 1), fixed),
            pl.BlockSpec((1, 1), fixed),
        ],
        out_specs=[
            pl.BlockSpec((_ROW_BLK, _H), vec),
            pl.BlockSpec((_ROW_BLK, 1), vec),
        ],
        out_shape=[
            jax.ShapeDtypeStruct((n, _H), jnp.float32),
            jax.ShapeDtypeStruct((n, 1), jnp.float32),
        ],
    )(agg, den, xl, xr, att, bias, g, b, wn[0], wn[1])


def _mlp_body(f_ref, w1_ref, b1_ref, w2_ref, b2_ref, o_ref):
    hd = jnp.dot(f_ref[...], w1_ref[...], preferred_element_type=jnp.float32)
    hd = jnp.maximum(hd + b1_ref[...], 0.0)
    o_ref[...] = jnp.dot(hd, w2_ref[...],
                         preferred_element_type=jnp.float32) + b2_ref[...]


def _mlp(flat, W1, b1, W2, b2):
    g, fin = flat.shape
    hid = W1.shape[1]
    c = W2.shape[1]
    return pl.pallas_call(
        _mlp_body,
        in_specs=[pl.BlockSpec(flat.shape, lambda: (0, 0)),
                  pl.BlockSpec(W1.shape, lambda: (0, 0)),
                  pl.BlockSpec((1, hid), lambda: (0, 0)),
                  pl.BlockSpec(W2.shape, lambda: (0, 0)),
                  pl.BlockSpec((1, c), lambda: (0, 0))],
        out_specs=pl.BlockSpec((g, c), lambda: (0, 0)),
        out_shape=jax.ShapeDtypeStruct((g, c), jnp.float32),
    )(flat, W1, b1.reshape(1, hid), W2, b2.reshape(1, c))


_EK = 80
_NGRP = _EK // 16
_NCHUNK = 125
_EGRP = _NCHUNK * _NGRP


def _edge_sc_call(xl, xr, att, idx2):
    mesh = plsc.VectorSubcoreMesh(core_axis_name="c", subcore_axis_name="s")

    @functools.partial(
        pl.kernel,
        mesh=mesh,
        out_type=jax.ShapeDtypeStruct((2, 16, _EGRP, 16), jnp.float32),
        scratch_types=[
            pltpu.VMEM((2, _EK), jnp.int32),
            pltpu.VMEM((_EK, _H), jnp.float32),
            pltpu.VMEM((_EK, _H), jnp.float32),
            pltpu.VMEM((_EGRP, 16), jnp.float32),
            pltpu.VMEM((8, 16), jnp.float32),
            pltpu.SemaphoreType.DMA,
            pltpu.SemaphoreType.DMA,
        ],
    )
    def body(xl_hbm, xr_hbm, att_hbm, idx_hbm, out_hbm,
             idx_v, xlv, xrv, eall, attv, sem1, sem2):
        c = lax.axis_index("c")
        s = lax.axis_index("s")
        pltpu.sync_copy(att_hbm, attv)

        iota16 = lax.iota(jnp.int32, 16)
        gdn = lax.GatherDimensionNumbers(
            offset_dims=(), collapsed_slice_dims=(0,), start_index_map=(0,))
        att_blks = [attv[blk] for blk in range(8)]

        def chunk(j, carry):
            pltpu.sync_copy(idx_hbm.at[c, s, j], idx_v)
            g1 = pltpu.async_copy(xl_hbm.at[idx_v.at[0]], xlv, sem1)
            g2 = pltpu.async_copy(xr_hbm.at[idx_v.at[1]], xrv, sem2)
            g1.wait()
            g2.wait()

            def group(g, carry2):
                def edge(e16, packed):
                    e = g * 16 + e16
                    acc = jnp.zeros((16,), jnp.float32)
                    for blk in range(8):
                        sl = pl.ds(blk * 16, 16)
                        m = xlv[e, sl] + xrv[e, sl]
                        m = jnp.where(m >= 0, m, 0.2 * m)
                        acc = acc + m * att_blks[blk]
                    tot = acc
                    for sh in (8, 4, 2, 1):
                        perm = (iota16 ^ sh).reshape(16, 1)
                        tot = tot + lax.gather(
                            tot, perm, gdn, slice_sizes=(1,),
                            mode=lax.GatherScatterMode.PROMISE_IN_BOUNDS)
                    ee = jnp.exp(tot)
                    return jnp.where(iota16 == e16, ee, packed)

                packed = lax.fori_loop(
                    0, 16, edge, jnp.zeros((16,), jnp.float32))
                eall[j * _NGRP + g] = packed
                return carry2

            lax.fori_loop(0, _NGRP, group, 0)
            return carry

        lax.fori_loop(0, _NCHUNK, chunk, 0)
        pltpu.sync_copy(eall, out_hbm.at[c, s])

    return body(xl, xr, att, idx2)


def _edge_aggregate(xl, xr, att, src, dst, evalid):
    n = xl.shape[0]
    srcg = jnp.where(evalid, src, 0).astype(jnp.int32)
    dstg = jnp.where(evalid, dst, 0).astype(jnp.int32)
    seg = jnp.where(evalid, dst, n).astype(jnp.int32)
    idx2 = jnp.stack([srcg.reshape(2, 16, _NCHUNK, _EK),
                      dstg.reshape(2, 16, _NCHUNK, _EK)], axis=3)
    ee = _edge_sc_call(xl, xr, att.reshape(8, 16), idx2).reshape(-1)
    ee = jnp.where(evalid, ee, 0.0)
    den = jax.ops.segment_sum(ee, seg, num_segments=n + 1)[:n]
    agg = jax.ops.segment_sum(xl[srcg] * ee[:, None], seg,
                              num_segments=n + 1)[:n]
    return agg, den


def _topk(score, batch, ei, valid, evalid):
    n = score.shape[0]
    bg = jnp.where(valid, batch, _G).astype(jnp.int32)
    order = jnp.lexsort((-score, bg))
    bs = bg[order]
    sizes = jax.ops.segment_sum(jnp.ones((n,), jnp.int32), bg,
                                num_segments=_G + 1)
    k = jnp.ceil(_RATIO * sizes.astype(jnp.float32)).astype(jnp.int32)
    starts = jnp.concatenate([jnp.zeros((1,), jnp.int32),
                              jnp.cumsum(sizes)[:-1].astype(jnp.int32)])
    rank = jnp.arange(n, dtype=jnp.int32) - starts[bs]
    keep = (rank < k[bs]) & (bs < _G)
    pos = jnp.cumsum(keep.astype(jnp.int32)) - 1
    mkeep = jnp.sum(keep.astype(jnp.int32))
    dest = jnp.where(keep, pos, n)
    newid = jnp.full((n,), -1, jnp.int32).at[order].set(
        jnp.where(keep, pos, -1).astype(jnp.int32))
    b_new = jnp.full((n + 1,), _G, jnp.int32).at[dest].set(bs)[:n]
    nsrc = newid[ei[0]]
    ndst = newid[ei[1]]
    ev_new = evalid & (nsrc >= 0) & (ndst >= 0)
    ei_new = jnp.stack([jnp.where(ev_new, nsrc, 0),
                        jnp.where(ev_new, ndst, 0)]).astype(jnp.int32)
    valid_new = jnp.arange(n, dtype=jnp.int32) < mkeep
    perm = jnp.zeros((n + 1,), jnp.int32).at[dest].set(order.astype(jnp.int32))[:n]
    return perm, mkeep, ei_new, b_new, valid_new, ev_new


def _gpool(h, b):
    add = jax.ops.segment_sum(h, b, num_segments=_G + 1)[:_G]
    mx = jax.ops.segment_max(h, b, num_segments=_G + 1)[:_G]
    mx = jnp.where(jnp.isfinite(mx), mx, 0.0)
    return jnp.concatenate([add, mx], axis=-1)


def kernel(x, edge_index, batch, params):
    n = x.shape[0]
    ei = edge_index.astype(jnp.int32)
    b = batch.astype(jnp.int32)
    valid = jnp.ones((n,), bool)
    evalid = jnp.ones((ei.shape[1],), bool)
    h = x
    flats = []
    for i in range(1, 5):
        p = params['conv%d' % i]
        xl, xr = _proj(h, p['Wl'], p['Wr'])
        agg, den = _edge_aggregate(xl, xr, p['att'], ei[0], ei[1], evalid)
        w = params['pool%d_w' % i]
        nrm = jnp.linalg.norm(w).reshape(1, 1)
        h, score = _combine(agg, den[:, None], xl, xr,
                            p['att'].reshape(_H, 1),
                            p['bias'].reshape(1, _H),
                            params['bn%d_g' % i].reshape(1, _H),
                            params['bn%d_b' % i].reshape(1, _H),
                            (w.reshape(_H, 1), nrm))
        score = score[:, 0]
        perm, mkeep, ei, b, valid, evalid = _topk(score, b, ei, valid, evalid)
        hp = h[perm] * score[perm][:, None]
        h = jnp.where((jnp.arange(n) < mkeep)[:, None], hp, 0.0)
        flats.append(_gpool(h, b))
    flat = jnp.concatenate(flats, axis=-1)
    return _mlp(flat, params['W1'], params['b1'], params['W2'], params['b2'])
